# baseline (speedup 1.0000x reference)
import functools

import jax
import jax.numpy as jnp
from jax.experimental import pallas as pl
from jax.experimental.pallas import tpu as pltpu

E = 8
K = 2
D = 768
H = 256
C_EXP = 100
C_PAD = 128
C_TOT = 800
N = 2048

TO = 512
NT = N // TO

_EPS = 2.220446049250313e-16


def _row(full, e):
    ridx = jax.lax.broadcasted_iota(jnp.int32, full.shape, 0)
    return jnp.sum(jnp.where(ridx == e, full, 0.0), axis=0, keepdims=True)


def _moe_kernel(x_ref, wg_ref, w1_ref, b1_ref, w2_ref, b2_ref, wm_ref,
                out_ref, og_s, wmc_s, gates_s, xb_s, w1_s, w2_s, wm_s,
                sem1, sem2, semm):
    s = pl.program_id(0)

    def _copies(e):
        return (pltpu.make_async_copy(w1_ref.at[e], w1_s.at[e], sem1.at[e]),
                pltpu.make_async_copy(w2_ref.at[e], w2_s.at[e], sem2.at[e]),
                pltpu.make_async_copy(wm_ref.at[e], wm_s.at[e], semm.at[e]))

    @pl.when(s == 0)
    def _gating():
        for e in range(E):
            for c in _copies(e):
                c.start()

        og_s[:] = jnp.zeros((N, E * C_PAD), jnp.bfloat16)
        wmc_s[:] = jnp.zeros((E * C_PAD, C_TOT), jnp.bfloat16)

        xt = x_ref[:]
        xb_s[:] = xt.astype(jnp.bfloat16)
        logits = jnp.dot(xt, wg_ref[:], preferred_element_type=jnp.float32)

        eidx = jax.lax.broadcasted_iota(jnp.int32, (N, E), 1)
        m1 = jnp.max(logits, axis=1, keepdims=True)
        a1 = jnp.argmax(logits, axis=1)[:, None]
        oh1 = (eidx == a1)
        masked = jnp.where(oh1, -jnp.inf, logits)
        m2 = jnp.max(masked, axis=1, keepdims=True)
        a2 = jnp.argmax(masked, axis=1)[:, None]
        oh2 = (eidx == a2)

        e2 = jnp.exp(m2 - m1)
        denom = 1.0 + e2
        gates_s[:] = (jnp.where(oh1, 1.0 / denom, 0.0)
                      + jnp.where(oh2, e2 / denom, 0.0))

    @pl.when(s < E)
    def _expert():
        e = s
        for c in _copies(e):
            c.wait()

        b1_row = _row(b1_ref[:], e)
        b2_row = _row(b2_ref[:], e)
        lidx = jax.lax.broadcasted_iota(jnp.int32, (N, E), 1)
        g_e = jnp.sum(jnp.where(lidx == e, gates_s[:], 0.0), axis=1,
                      keepdims=True)

        wmc_s[pl.ds(e * C_PAD, C_EXP), :] = wm_s[e].astype(jnp.bfloat16)

        h = jnp.dot(xb_s[:], w1_s[e].astype(jnp.bfloat16),
                    preferred_element_type=jnp.float32)
        h = jnp.maximum(h + b1_row, 0.0).astype(jnp.bfloat16)
        o = jnp.dot(h, w2_s[e].astype(jnp.bfloat16),
                    preferred_element_type=jnp.float32)
        o = (o + b2_row) * g_e
        og_s[:, pl.ds(e * C_PAD, C_EXP)] = o.astype(jnp.bfloat16)

    @pl.when(s >= E)
    def _mapper():
        t = s - E
        og = og_s[pl.ds(t * TO, TO), :]
        acc = jnp.dot(og, wmc_s[:], preferred_element_type=jnp.float32)
        out_ref[:] = jnp.where(acc == 0.0, jnp.float32(_EPS), acc)


@functools.partial(jax.jit, static_argnames=("interpret",))
def _moe(x, w_gate, W1, b1, W2, b2, Wm, interpret=False):
    full = lambda *sh: pl.BlockSpec(sh, lambda s: (0,) * len(sh))
    anyspec = pl.BlockSpec(memory_space=pl.ANY)
    return pl.pallas_call(
        _moe_kernel,
        grid=(E + NT,),
        in_specs=[
            full(N, D),
            full(D, E),
            anyspec,
            full(E, H),
            anyspec,
            full(E, C_EXP),
            anyspec,
        ],
        out_specs=pl.BlockSpec(
            (TO, C_TOT), lambda s: (jnp.clip(s - E, 0, NT - 1), 0)),
        out_shape=jax.ShapeDtypeStruct((N, C_TOT), jnp.float32),
        scratch_shapes=[
            pltpu.VMEM((N, E * C_PAD), jnp.bfloat16),
            pltpu.VMEM((E * C_PAD, C_TOT), jnp.bfloat16),
            pltpu.VMEM((N, E), jnp.float32),
            pltpu.VMEM((N, D), jnp.bfloat16),
            pltpu.VMEM((E, D, H), jnp.float32),
            pltpu.VMEM((E, H, C_EXP), jnp.float32),
            pltpu.VMEM((E, C_EXP, C_TOT), jnp.float32),
            pltpu.SemaphoreType.DMA((E,)),
            pltpu.SemaphoreType.DMA((E,)),
            pltpu.SemaphoreType.DMA((E,)),
        ],
        compiler_params=pltpu.CompilerParams(
            dimension_semantics=("arbitrary",)),
        interpret=interpret,
    )(x, w_gate, W1, b1, W2, b2, Wm)


def kernel(x, labels, w_gate, W1, b1, W2, b2, Wm):
    return _moe(x, w_gate, W1, b1, W2, b2, Wm)


# speedup vs baseline: 1.0518x; 1.0518x over previous
import functools

import jax
import jax.numpy as jnp
from jax.experimental import pallas as pl
from jax.experimental.pallas import tpu as pltpu

E = 8
K = 2
D = 768
H = 256
C_EXP = 100
C_PAD = 128
C_TOT = 800
N = 2048

_EPS = 2.220446049250313e-16


def _moe_kernel(x_ref, wg_ref, w1_ref, b1_ref, w2_ref, b2_ref, wm_ref,
                out_ref, w1c_s, b1c_s, w2p_s, b2p_s, wmc_s):
    @pl.when(pl.program_id(0) == 0)
    def _prep():
        zlane = jnp.zeros((H, C_PAD - C_EXP), dtype=jnp.bfloat16)
        zrow = jnp.zeros((C_PAD - C_EXP, C_TOT), dtype=jnp.bfloat16)
        for e in range(E):
            w1c_s[:, e * H:(e + 1) * H] = w1_ref[e].astype(jnp.bfloat16)
            b1c_s[0:1, e * H:(e + 1) * H] = b1_ref[e:e + 1, :]
            w2p_s[e, :, :C_EXP] = w2_ref[e].astype(jnp.bfloat16)
            w2p_s[e, :, C_EXP:] = zlane
            b2p_s[e:e + 1, :C_EXP] = b2_ref[e:e + 1, :]
            b2p_s[e:e + 1, C_EXP:] = jnp.zeros((1, C_PAD - C_EXP), jnp.float32)
            wmc_s[e * C_PAD:e * C_PAD + C_EXP, :] = wm_ref[e].astype(jnp.bfloat16)
            wmc_s[e * C_PAD + C_EXP:(e + 1) * C_PAD, :] = zrow

    xt = x_ref[:]
    t = xt.shape[0]
    logits = jnp.dot(xt, wg_ref[:], preferred_element_type=jnp.float32)

    eidx = jax.lax.broadcasted_iota(jnp.int32, (t, E), 1)
    m1 = jnp.max(logits, axis=1, keepdims=True)
    a1 = jnp.argmax(logits, axis=1)[:, None]
    oh1 = (eidx == a1)
    masked = jnp.where(oh1, -jnp.inf, logits)
    m2 = jnp.max(masked, axis=1, keepdims=True)
    a2 = jnp.argmax(masked, axis=1)[:, None]
    oh2 = (eidx == a2)

    e2 = jnp.exp(m2 - m1)
    denom = 1.0 + e2
    g1 = 1.0 / denom
    g2 = e2 / denom
    gates = jnp.where(oh1, g1, 0.0) + jnp.where(oh2, g2, 0.0)

    xb = xt.astype(jnp.bfloat16)
    hc = jnp.dot(xb, w1c_s[:], preferred_element_type=jnp.float32)
    hc = jnp.maximum(hc + b1c_s[:], 0.0).astype(jnp.bfloat16)

    o_blocks = []
    for e in range(E):
        o_e = jnp.dot(hc[:, e * H:(e + 1) * H], w2p_s[e],
                      preferred_element_type=jnp.float32)
        o_e = (o_e + b2p_s[e][None, :]) * gates[:, e][:, None]
        o_blocks.append(o_e.astype(jnp.bfloat16))
    og = jnp.concatenate(o_blocks, axis=1)

    acc = jnp.dot(og, wmc_s[:], preferred_element_type=jnp.float32)
    acc = jnp.where(acc == 0.0, jnp.float32(_EPS), acc)
    out_ref[:] = acc


@functools.partial(jax.jit, static_argnames=("interpret", "T"))
def _moe(x, w_gate, W1, b1, W2, b2, Wm, interpret=False, T=512):
    grid = (N // T,)
    full = lambda *s: pl.BlockSpec(s, lambda i: (0,) * len(s))
    return pl.pallas_call(
        _moe_kernel,
        grid=grid,
        in_specs=[
            pl.BlockSpec((T, D), lambda i: (i, 0)),
            full(D, E),
            full(E, D, H),
            full(E, H),
            full(E, H, C_EXP),
            full(E, C_EXP),
            full(E, C_EXP, C_TOT),
        ],
        out_specs=pl.BlockSpec((T, C_TOT), lambda i: (i, 0)),
        out_shape=jax.ShapeDtypeStruct((N, C_TOT), jnp.float32),
        scratch_shapes=[
            pltpu.VMEM((D, E * H), jnp.bfloat16),
            pltpu.VMEM((1, E * H), jnp.float32),
            pltpu.VMEM((E, H, C_PAD), jnp.bfloat16),
            pltpu.VMEM((E, C_PAD), jnp.float32),
            pltpu.VMEM((E * C_PAD, C_TOT), jnp.bfloat16),
        ],
        compiler_params=pltpu.CompilerParams(
            dimension_semantics=("arbitrary",)),
        interpret=interpret,
    )(x, w_gate, W1, b1, W2, b2, Wm)


def kernel(x, labels, w_gate, W1, b1, W2, b2, Wm):
    return _moe(x, w_gate, W1, b1, W2, b2, Wm)
